# Initial kernel scaffold; baseline (speedup 1.0000x reference)
import jax
import jax.numpy as jnp
from jax.experimental import pallas as pl


def kernel(x, table):
    raise NotImplementedError("write your pallas kernel here")


# speedup vs baseline: 1.1800x; 1.1800x over previous
import functools

import jax
import jax.numpy as jnp
from jax import lax
from jax.experimental import pallas as pl
from jax.experimental.pallas import tpu as pltpu
from jax.experimental.pallas import tpu_sc as plsc

NUM_ROWS = 2600000
BATCH = 16384
NUM_FIELDS = 26

_info = plsc.get_sparse_core_info()
NC, NS, L = _info.num_cores, _info.num_subcores, _info.num_lanes
NW = NC * NS
B_PER_W = BATCH // NW
E_PER_W = B_PER_W * NUM_FIELDS
CHUNK = 128
N_DMA = E_PER_W // CHUNK
N_OUT_CHUNKS = B_PER_W // L


def _sc_kernel(x_hbm, table_hbm, out_hbm, idx_v, vals_v, out_v, sem):
    wid = lax.axis_index("s") * NC + lax.axis_index("c")

    pltpu.sync_copy(x_hbm.at[wid], idx_v)

    def fire(j, c):
        dst = vals_v.at[pl.ds(pl.multiple_of(j * CHUNK, CHUNK), CHUNK)]
        pltpu.async_copy(table_hbm.at[idx_v.at[j]], dst, sem)
        return c

    lax.fori_loop(0, N_DMA, fire, 0)

    def drain(j, c):
        dst = vals_v.at[pl.ds(pl.multiple_of(j * CHUNK, CHUNK), CHUNK)]
        pltpu.make_async_copy(table_hbm.at[idx_v.at[j]], dst, sem).wait()
        return c

    lax.fori_loop(0, N_DMA, drain, 0)

    lane = lax.iota(jnp.int32, L)
    lane26 = lane * NUM_FIELDS

    def reduce_chunk(i, c):
        base = i * (L * NUM_FIELDS)
        acc = jnp.zeros((L,), jnp.float32)
        for f in range(NUM_FIELDS):
            acc = acc + plsc.load_gather(vals_v, [lane26 + (base + f)])
        out_v[pl.ds(pl.multiple_of(i * L, L), L)] = acc
        return c

    lax.fori_loop(0, N_OUT_CHUNKS, reduce_chunk, 0)

    off = pl.multiple_of(wid * B_PER_W, B_PER_W)
    pltpu.sync_copy(out_v, out_hbm.at[pl.ds(off, B_PER_W)])


@jax.jit
def kernel(x, table):
    x_blk = x.reshape(NW, N_DMA, CHUNK)
    table_flat = table.reshape(NUM_ROWS)
    mesh = plsc.VectorSubcoreMesh(core_axis_name="c", subcore_axis_name="s")
    out = pl.kernel(
        _sc_kernel,
        mesh=mesh,
        compiler_params=pltpu.CompilerParams(needs_layout_passes=False),
        out_type=jax.ShapeDtypeStruct((BATCH,), jnp.float32),
        scratch_types=[
            pltpu.VMEM((N_DMA, CHUNK), jnp.int32),
            pltpu.VMEM((E_PER_W,), jnp.float32),
            pltpu.VMEM((B_PER_W,), jnp.float32),
            pltpu.SemaphoreType.DMA,
        ],
    )(x_blk, table_flat)
    return out.reshape(BATCH, 1)
